# Initial kernel scaffold; baseline (speedup 1.0000x reference)
import jax
import jax.numpy as jnp
from jax.experimental import pallas as pl


def kernel(features, edge_index, W1, b1, W2, b2, Wd1, bd1, Wd2, bd2):
    raise NotImplementedError("write your pallas kernel here")


# speedup vs baseline: 6.9208x; 6.9208x over previous
import functools

import jax
import jax.numpy as jnp
from jax import lax
from jax.experimental import pallas as pl
from jax.experimental.pallas import tpu as pltpu
from jax.experimental.pallas import tpu_sc as plsc

N = 10000
E = 320000
D = 128

NC = 2
NS = 16
NW = NC * NS
EPW = E // NW
CH = 80
NCHUNK = EPW // CH
GC = 25
NG = NCHUNK // GC
FR = 624
CR = 208
ZCR = 48
TAIL = N - NS * FR


def _zero_f32(ref):
  shape = ref.shape
  total = 1
  for s in shape:
    total *= s
  flat_chunks = total // 16
  if len(shape) == 1:
    def body(i, _):
      ref[pl.ds(i * 16, 16)] = jnp.zeros((16,), jnp.float32)
      return 0
    lax.fori_loop(0, flat_chunks, body, 0)
  else:
    rows, cols = shape
    def body(i, _):
      for k in range(cols // 16):
        ref[i, pl.ds(k * 16, 16)] = jnp.zeros((16,), jnp.float32)
      return 0
    lax.fori_loop(0, rows, body, 0)


def _make_sc_agg(with_deg):
  mesh = plsc.VectorSubcoreMesh(core_axis_name="c", subcore_axis_name="s")
  out_type = [jax.ShapeDtypeStruct((NC, N, D), jnp.float32)]
  if with_deg:
    out_type.append(jax.ShapeDtypeStruct((NC, N), jnp.float32))

  scratch = [
      pltpu.VMEM_SHARED((N, D), jnp.float32),
      pltpu.VMEM((GC, CH), jnp.int32),
      pltpu.VMEM((GC, CH), jnp.int32),
      pltpu.VMEM((CH, D), jnp.float32),
      pltpu.VMEM((ZCR, D), jnp.float32),
      pltpu.SemaphoreType.DMA,
  ]
  if with_deg:
    scratch += [
        pltpu.VMEM_SHARED((N,), jnp.float32),
        pltpu.VMEM((CH,), jnp.float32),
        pltpu.VMEM((FR,), jnp.float32),
    ]

  def body(x_hbm, src_hbm, dst_hbm, *rest):
    if with_deg:
      (sums_hbm, deg_hbm, acc, src_v, dst_v, rows, zbuf, sem,
       dacc, ones, dzero) = rest
    else:
      (sums_hbm, acc, src_v, dst_v, rows, zbuf, sem) = rest
    c = lax.axis_index("c")
    s = lax.axis_index("s")
    wid = c * NS + s

    _zero_f32(zbuf)
    for t in range(FR // ZCR):
      r0 = s * FR + t * ZCR
      pltpu.sync_copy(zbuf, acc.at[pl.ds(r0, ZCR), :])
    @pl.when(s == NS - 1)
    def _():
      pltpu.sync_copy(zbuf.at[pl.ds(0, TAIL), :], acc.at[pl.ds(NS * FR, TAIL), :])
    if with_deg:
      _zero_f32(ones)
      def obody(i, _):
        ones[pl.ds(i * 16, 16)] = jnp.ones((16,), jnp.float32)
        return 0
      lax.fori_loop(0, CH // 16, obody, 0)
      _zero_f32(dzero)
      pltpu.sync_copy(dzero, dacc.at[pl.ds(s * FR, FR)])
      @pl.when(s == NS - 1)
      def _():
        pltpu.sync_copy(dzero.at[pl.ds(0, TAIL)], dacc.at[pl.ds(NS * FR, TAIL)])
    plsc.subcore_barrier()

    def group(g, _):
      pltpu.sync_copy(src_hbm.at[wid, g], src_v)
      pltpu.sync_copy(dst_hbm.at[wid, g], dst_v)
      def chunk(j, _):
        pltpu.async_copy(x_hbm.at[src_v.at[j]], rows, sem).wait()
        pltpu.sync_copy(rows, acc.at[dst_v.at[j]], add=True)
        if with_deg:
          pltpu.sync_copy(ones, dacc.at[dst_v.at[j]], add=True)
        return 0
      lax.fori_loop(0, GC, chunk, 0)
      return 0
    lax.fori_loop(0, NG, group, 0)

    plsc.subcore_barrier()

    for t in range(FR // CR):
      r0 = s * FR + t * CR
      pltpu.sync_copy(acc.at[pl.ds(r0, CR), :], sums_hbm.at[c, pl.ds(r0, CR), :])
    @pl.when(s == NS - 1)
    def _():
      r0 = NS * FR
      pltpu.sync_copy(acc.at[pl.ds(r0, TAIL), :], sums_hbm.at[c, pl.ds(r0, TAIL), :])
    if with_deg:
      @pl.when(s == 0)
      def _():
        pltpu.sync_copy(dacc, deg_hbm.at[c])

  return pl.kernel(body, out_type=out_type, mesh=mesh, scratch_types=scratch)


_sc_agg_deg = _make_sc_agg(True)
_sc_agg = _make_sc_agg(False)


BN = 1000


def _tc1_body(x_ref, s0_ref, s1_ref, d0_ref, d1_ref, w_ref, b_ref, o_ref):
  deg = jnp.maximum(d0_ref[...] + d1_ref[...], 1.0)
  agg = (s0_ref[...] + s1_ref[...]) / deg
  h = (x_ref[...] + agg) @ w_ref[...] + b_ref[...]
  o_ref[...] = jnp.maximum(h, 0.0)


def _tc1(x, s0, s1, d0, d1, w, b):
  row = pl.BlockSpec((BN, D), lambda i: (i, 0))
  col = pl.BlockSpec((BN, 1), lambda i: (i, 0))
  full = pl.BlockSpec((D, D), lambda i: (0, 0))
  bias = pl.BlockSpec((1, D), lambda i: (0, 0))
  return pl.pallas_call(
      _tc1_body,
      grid=(N // BN,),
      in_specs=[row, row, row, col, col, full, bias],
      out_specs=row,
      out_shape=jax.ShapeDtypeStruct((N, D), jnp.float32),
  )(x, s0, s1, d0, d1, w, b)


def _tc2_body(h_ref, s0_ref, s1_ref, d0_ref, d1_ref, w2_ref, b2_ref,
              wd1_ref, bd1_ref, wd2_ref, bd2_ref, hg_ref, rec_ref, acc_ref):
  i = pl.program_id(0)

  @pl.when(i == 0)
  def _():
    acc_ref[...] = jnp.zeros_like(acc_ref)

  deg = jnp.maximum(d0_ref[...] + d1_ref[...], 1.0)
  agg = (s0_ref[...] + s1_ref[...]) / deg
  h2 = jnp.maximum((h_ref[...] + agg) @ w2_ref[...] + b2_ref[...], 0.0)
  acc_ref[...] += jnp.sum(h2, axis=0, keepdims=True)

  @pl.when(i == pl.num_programs(0) - 1)
  def _():
    hg = acc_ref[...] * (1.0 / N)
    hg_ref[...] = hg
    r = jnp.maximum(hg @ wd1_ref[...] + bd1_ref[...], 0.0)
    rec_ref[...] = r @ wd2_ref[...] + bd2_ref[...]


def _tc2(h, s0, s1, d0, d1, w2, b2, wd1, bd1, wd2, bd2):
  row = pl.BlockSpec((BN, D), lambda i: (i, 0))
  col = pl.BlockSpec((BN, 1), lambda i: (i, 0))
  full = pl.BlockSpec((D, D), lambda i: (0, 0))
  bias = pl.BlockSpec((1, D), lambda i: (0, 0))
  out = pl.BlockSpec((1, D), lambda i: (0, 0))
  return pl.pallas_call(
      _tc2_body,
      grid=(N // BN,),
      in_specs=[row, row, row, col, col, full, bias, full, bias, full, bias],
      out_specs=[out, out],
      out_shape=[jax.ShapeDtypeStruct((1, D), jnp.float32),
                 jax.ShapeDtypeStruct((1, D), jnp.float32)],
      scratch_shapes=[pltpu.VMEM((1, D), jnp.float32)],
  )(h, s0, s1, d0, d1, w2, b2, wd1, bd1, wd2, bd2)


@jax.jit
def kernel(features, edge_index, W1, b1, W2, b2, Wd1, bd1, Wd2, bd2):
  src = edge_index[0].reshape(NW, NG, GC, CH)
  dst = edge_index[1].reshape(NW, NG, GC, CH)

  sums1, deg = _sc_agg_deg(features, src, dst)
  d0 = deg[0].reshape(N, 1)
  d1 = deg[1].reshape(N, 1)
  h1 = _tc1(features, sums1[0], sums1[1], d0, d1, W1, b1.reshape(1, D))

  (sums2,) = _sc_agg(h1, src, dst)
  hg, rec = _tc2(h1, sums2[0], sums2[1], d0, d1, W2, b2.reshape(1, D),
                 Wd1, bd1.reshape(1, D), Wd2, bd2.reshape(1, D))
  return (hg, rec)
